# Initial kernel scaffold; baseline (speedup 1.0000x reference)
import jax
import jax.numpy as jnp
from jax.experimental import pallas as pl


def kernel(gt_cls, gt_offsets, gt_segments, segments_label, segments_mask, fpn_masks, out_cls_logits, out_offsets, out_rois, out_scores, out_roimask, cls_log, com_log, cls_gt, cls_node, com_gt, com_node):
    raise NotImplementedError("write your pallas kernel here")


# speedup vs baseline: 3.7765x; 3.7765x over previous
import functools

import jax
import jax.numpy as jnp
from jax import lax
from jax.experimental import pallas as pl
from jax.experimental.pallas import tpu as pltpu

_B = 2
_T = 2304
_NP = 1024
_NG = 32
_NC1 = 21
_INIT_LOSS_NORM = 100.0
_LOSS_WEIGHT = 1.0
_FG_IOU = 0.7
_BG_IOU = 0.01
_COM_IOU = 0.3
_SAMPLE_RATIO = 6.0
_EPS = 1e-8


def _loss_kernel(
    gt_cls_ref,
    fpn_ref,
    logits_ref,
    off_l_ref,
    off_r_ref,
    goff_l_ref,
    goff_r_ref,
    gseg_l_ref,
    gseg_r_ref,
    glab_ref,
    segmask_ref,
    roi_l_ref,
    roi_r_ref,
    scores_ref,
    cls_log_ref,
    com_log_ref,
    cls_gt_ref,
    com_gt_ref,
    cls_node_ref,
    com_node_ref,
    out_ref,
):
    f32 = jnp.float32

    row = lax.broadcasted_iota(jnp.int32, (_NP, _NP), 0)
    col = lax.broadcasted_iota(jnp.int32, (_NP, _NP), 1)
    tril = (col <= row).astype(f32)

    cls_nll_sum = f32(0.0)
    cls_cnt = f32(0.0)
    com_nll_sum = f32(0.0)
    com_cnt = f32(0.0)

    for j in range(_B):
        gseg_l = gseg_l_ref[j]
        gseg_r = gseg_r_ref[j]
        segmask = segmask_ref[j]
        roi_l = roi_l_ref[j]
        roi_r = roi_r_ref[j]

        min_left = jnp.minimum(gseg_l, roi_l)
        max_left = jnp.maximum(gseg_l, roi_l)
        min_right = jnp.minimum(gseg_r, roi_r)
        max_right = jnp.maximum(gseg_r, roi_r)
        ious_mat = (min_right - max_left) / (max_right - min_left)
        ious_mat = jnp.where(segmask > 0.0, ious_mat, -jnp.inf)

        ious = jnp.max(ious_mat, axis=1, keepdims=True)
        kiota = lax.broadcasted_iota(jnp.int32, (_NP, _NG), 1)
        is_max = ious_mat == ious
        amin = jnp.min(jnp.where(is_max, kiota, _NG), axis=1, keepdims=True)
        onehot = kiota == amin
        oh_f = onehot.astype(f32)

        lab = jnp.sum(jnp.where(onehot, glab_ref[j], 0), axis=1, keepdims=True)
        pos = ious > _FG_IOU
        pos_f = pos.astype(f32)
        lab = lab * pos.astype(jnp.int32)
        num_pos = jnp.sum(pos_f)

        def masked_cos(gt_feat, node_feat):
            gnorm = jnp.sqrt(jnp.sum(gt_feat * gt_feat, axis=1, keepdims=True))
            gn = gt_feat / jnp.clip(gnorm, _EPS, None)
            s = lax.dot_general(node_feat, gn, (((1,), (1,)), ((), ())),
                                preferred_element_type=f32)
            nnorm = jnp.sqrt(jnp.sum(node_feat * node_feat, axis=1, keepdims=True))
            cos = jnp.sum(s * oh_f, axis=1, keepdims=True) / jnp.clip(nnorm, _EPS, None)
            return 1.0 - (cos + 1.0) / 2.0

        cls_dist = masked_cos(cls_gt_ref[j], cls_node_ref[j])
        com_dist = masked_cos(com_gt_ref[j], com_node_ref[j])
        del cls_dist, com_dist

        scores = scores_ref[j]
        pro_ok = (scores > 0.0) & (ious > 0.0)
        bg_pro = (ious < _BG_IOU) & pro_ok
        com_pro = (ious < _COM_IOU) & pro_ok
        procols = jnp.concatenate(
            [bg_pro.astype(f32), com_pro.astype(f32)], axis=1)
        csum = lax.dot_general(tril, procols, (((1,), (0,)), ((), ())),
                               preferred_element_type=f32)
        bg_sel = bg_pro & (csum[:, 0:1] <= num_pos)
        com_sel = com_pro & (csum[:, 1:2] <= jnp.maximum(1.0, _SAMPLE_RATIO * num_pos))
        sel = (pos | bg_sel).astype(f32)
        sel_com = (pos | com_sel).astype(f32)

        ciota = lax.broadcasted_iota(jnp.int32, (_NP, _NC1), 1)
        lab_oh = (ciota == lab).astype(f32)

        def nll_at(logit):
            mx = jnp.max(logit, axis=1, keepdims=True)
            sh = logit - mx
            lse = jnp.log(jnp.sum(jnp.exp(sh), axis=1, keepdims=True))
            ls = sh - lse
            return -jnp.sum(ls * lab_oh, axis=1, keepdims=True)

        cls_nll_sum += jnp.sum(nll_at(cls_log_ref[j]) * sel)
        cls_cnt += jnp.sum(sel)
        com_nll_sum += jnp.sum(nll_at(com_log_ref[j]) * sel_com)
        com_cnt += jnp.sum(sel_com)

    prop_loss = cls_nll_sum / cls_cnt + 0.5 * (com_nll_sum / com_cnt)

    fpn = fpn_ref[...]
    gt_cls = gt_cls_ref[...]
    gt_target = (gt_cls > 0).astype(f32)
    pos_mask = gt_target * fpn
    num_pos_bt = jnp.sum(pos_mask)
    loss_norm = 0.9 * _INIT_LOSS_NORM + 0.1 * jnp.maximum(num_pos_bt, 1.0)

    x = logits_ref[...]
    p = jax.nn.sigmoid(x)
    ce = jnp.maximum(x, 0.0) - x * gt_target + jnp.log1p(jnp.exp(-jnp.abs(x)))
    p_t = p * gt_target + (1.0 - p) * (1.0 - gt_target)
    focal = ce * (1.0 - p_t) ** 2.0
    focal = focal * (0.25 * gt_target + 0.75 * (1.0 - gt_target))
    cls_loss = jnp.sum(focal * fpn) / loss_norm

    lp = off_l_ref[...]
    rp = off_r_ref[...]
    lg = goff_l_ref[...]
    rg = goff_r_ref[...]
    intsctk = jnp.minimum(lp, lg) + jnp.minimum(rp, rg)
    unionk = (lp + rp) + (lg + rg) - intsctk
    iouk = intsctk / jnp.clip(unionk, _EPS, None)
    len_c = jnp.maximum(lp, lg) + jnp.maximum(rp, rg)
    miouk = iouk - (len_c - unionk) / jnp.clip(len_c, _EPS, None)
    reg_loss = jnp.sum((1.0 - miouk) * pos_mask) / loss_norm

    out_ref[0, 0] = cls_loss + reg_loss * _LOSS_WEIGHT + prop_loss


@functools.partial(jax.jit, static_argnames=())
def _run(gt_cls, fpn_f, logits, off_l, off_r, goff_l, goff_r,
         gseg_l, gseg_r, glab, segmask, roi_l, roi_r, scores,
         cls_log, com_log, cls_gt, com_gt, cls_node, com_node):
    out = pl.pallas_call(
        _loss_kernel,
        out_shape=jax.ShapeDtypeStruct((1, 1), jnp.float32),
        out_specs=pl.BlockSpec(memory_space=pltpu.SMEM),
    )(gt_cls, fpn_f, logits, off_l, off_r, goff_l, goff_r,
      gseg_l, gseg_r, glab, segmask, roi_l, roi_r, scores,
      cls_log, com_log, cls_gt, com_gt, cls_node, com_node)
    return out[0, 0]


def kernel(gt_cls, gt_offsets, gt_segments, segments_label, segments_mask,
           fpn_masks, out_cls_logits, out_offsets, out_rois, out_scores,
           out_roimask, cls_log, com_log, cls_gt, cls_node, com_gt, com_node):
    f32 = jnp.float32
    return _run(
        gt_cls.astype(jnp.int32),
        fpn_masks.astype(f32),
        out_cls_logits.astype(f32),
        out_offsets[..., 0].astype(f32),
        out_offsets[..., 1].astype(f32),
        gt_offsets[..., 0].astype(f32),
        gt_offsets[..., 1].astype(f32),
        gt_segments[..., 0].reshape(_B, 1, _NG).astype(f32),
        gt_segments[..., 1].reshape(_B, 1, _NG).astype(f32),
        segments_label.reshape(_B, 1, _NG).astype(jnp.int32),
        segments_mask.reshape(_B, 1, _NG).astype(f32),
        out_rois[..., 1].reshape(_B, _NP, 1).astype(f32),
        out_rois[..., 2].reshape(_B, _NP, 1).astype(f32),
        out_scores.reshape(_B, _NP, 1).astype(f32),
        cls_log.astype(f32),
        com_log.astype(f32),
        cls_gt.astype(f32),
        com_gt.astype(f32),
        cls_node.astype(f32),
        com_node.astype(f32),
    )


# speedup vs baseline: 7.6235x; 2.0187x over previous
import functools

import jax
import jax.numpy as jnp
from jax import lax
from jax.experimental import pallas as pl
from jax.experimental.pallas import tpu as pltpu

_B = 2
_T = 2304
_NP = 1024
_NG = 32
_NC1 = 21
_INIT_LOSS_NORM = 100.0
_LOSS_WEIGHT = 1.0
_FG_IOU = 0.7
_BG_IOU = 0.01
_COM_IOU = 0.3
_SAMPLE_RATIO = 6.0
_EPS = 1e-8


def _lane_cumsum(x):
    n = x.shape[-1]
    s = 1
    while s < n:
        shifted = jnp.concatenate(
            [jnp.zeros(x.shape[:-1] + (s,), x.dtype), x[..., : n - s]], axis=-1)
        x = x + shifted
        s *= 2
    return x


def _loss_kernel(
    gt_cls_ref,
    fpn_ref,
    logits_ref,
    off_l_ref,
    off_r_ref,
    goff_l_ref,
    goff_r_ref,
    gseg_l_ref,
    gseg_r_ref,
    glab_ref,
    segmask_ref,
    roi_l_ref,
    roi_r_ref,
    scores_ref,
    cls_log_ref,
    com_log_ref,
    out_ref,
):
    f32 = jnp.float32

    cls_nll_sum = f32(0.0)
    cls_cnt = f32(0.0)
    com_nll_sum = f32(0.0)
    com_cnt = f32(0.0)

    for j in range(_B):
        gseg_l = gseg_l_ref[j]
        gseg_r = gseg_r_ref[j]
        segmask = segmask_ref[j]
        roi_l = roi_l_ref[j]
        roi_r = roi_r_ref[j]

        min_left = jnp.minimum(gseg_l, roi_l)
        max_left = jnp.maximum(gseg_l, roi_l)
        min_right = jnp.minimum(gseg_r, roi_r)
        max_right = jnp.maximum(gseg_r, roi_r)
        ious_mat = (min_right - max_left) / (max_right - min_left)
        ious_mat = jnp.where(segmask > 0.0, ious_mat, -jnp.inf)

        ious = jnp.max(ious_mat, axis=0, keepdims=True)
        kiota = lax.broadcasted_iota(jnp.int32, (_NG, _NP), 0)
        is_max = ious_mat == ious
        amin = jnp.min(jnp.where(is_max, kiota, _NG), axis=0, keepdims=True)
        onehot = kiota == amin

        lab = jnp.sum(jnp.where(onehot, glab_ref[j], 0), axis=0, keepdims=True)
        pos = ious > _FG_IOU
        pos_f = pos.astype(f32)
        lab = lab * pos.astype(jnp.int32)
        num_pos = jnp.sum(pos_f)

        scores = scores_ref[j]
        pro_ok = (scores > 0.0) & (ious > 0.0)
        bg_pro = (ious < _BG_IOU) & pro_ok
        com_pro = (ious < _COM_IOU) & pro_ok
        procols = jnp.concatenate(
            [bg_pro.astype(f32), com_pro.astype(f32)], axis=0)
        csum = _lane_cumsum(procols)
        bg_sel = bg_pro & (csum[0:1, :] <= num_pos)
        com_sel = com_pro & (csum[1:2, :] <= jnp.maximum(1.0, _SAMPLE_RATIO * num_pos))
        sel = (pos | bg_sel).astype(f32)
        sel_com = (pos | com_sel).astype(f32)

        ciota = lax.broadcasted_iota(jnp.int32, (_NC1, _NP), 0)
        lab_oh = (ciota == lab).astype(f32)

        def nll_at(logit):
            mx = jnp.max(logit, axis=0, keepdims=True)
            sh = logit - mx
            lse = jnp.log(jnp.sum(jnp.exp(sh), axis=0, keepdims=True))
            ls = sh - lse
            return -jnp.sum(ls * lab_oh, axis=0, keepdims=True)

        cls_nll_sum += jnp.sum(nll_at(cls_log_ref[j]) * sel)
        cls_cnt += jnp.sum(sel)
        com_nll_sum += jnp.sum(nll_at(com_log_ref[j]) * sel_com)
        com_cnt += jnp.sum(sel_com)

    prop_loss = cls_nll_sum / cls_cnt + 0.5 * (com_nll_sum / com_cnt)

    fpn = fpn_ref[...]
    gt_cls = gt_cls_ref[...]
    gt_target = (gt_cls > 0).astype(f32)
    pos_mask = gt_target * fpn
    num_pos_bt = jnp.sum(pos_mask)
    loss_norm = 0.9 * _INIT_LOSS_NORM + 0.1 * jnp.maximum(num_pos_bt, 1.0)

    x = logits_ref[...]
    p = jax.nn.sigmoid(x)
    ce = jnp.maximum(x, 0.0) - x * gt_target + jnp.log1p(jnp.exp(-jnp.abs(x)))
    p_t = p * gt_target + (1.0 - p) * (1.0 - gt_target)
    omp = 1.0 - p_t
    focal = ce * (omp * omp)
    focal = focal * (0.25 * gt_target + 0.75 * (1.0 - gt_target))
    cls_loss = jnp.sum(focal * fpn) / loss_norm

    lp = off_l_ref[...]
    rp = off_r_ref[...]
    lg = goff_l_ref[...]
    rg = goff_r_ref[...]
    intsctk = jnp.minimum(lp, lg) + jnp.minimum(rp, rg)
    unionk = (lp + rp) + (lg + rg) - intsctk
    iouk = intsctk / jnp.clip(unionk, _EPS, None)
    len_c = jnp.maximum(lp, lg) + jnp.maximum(rp, rg)
    miouk = iouk - (len_c - unionk) / jnp.clip(len_c, _EPS, None)
    reg_loss = jnp.sum((1.0 - miouk) * pos_mask) / loss_norm

    out_ref[0, 0] = cls_loss + reg_loss * _LOSS_WEIGHT + prop_loss


@functools.partial(jax.jit, static_argnames=())
def _run(gt_cls, fpn_f, logits, off_l, off_r, goff_l, goff_r,
         gseg_l, gseg_r, glab, segmask, roi_l, roi_r, scores,
         cls_log_t, com_log_t):
    out = pl.pallas_call(
        _loss_kernel,
        out_shape=jax.ShapeDtypeStruct((1, 1), jnp.float32),
        out_specs=pl.BlockSpec(memory_space=pltpu.SMEM),
    )(gt_cls, fpn_f, logits, off_l, off_r, goff_l, goff_r,
      gseg_l, gseg_r, glab, segmask, roi_l, roi_r, scores,
      cls_log_t, com_log_t)
    return out[0, 0]


def kernel(gt_cls, gt_offsets, gt_segments, segments_label, segments_mask,
           fpn_masks, out_cls_logits, out_offsets, out_rois, out_scores,
           out_roimask, cls_log, com_log, cls_gt, cls_node, com_gt, com_node):
    f32 = jnp.float32
    bt = (36, 128)
    return _run(
        gt_cls.astype(jnp.int32).reshape(bt),
        fpn_masks.astype(f32).reshape(bt),
        out_cls_logits.astype(f32).reshape(bt),
        out_offsets[..., 0].astype(f32).reshape(bt),
        out_offsets[..., 1].astype(f32).reshape(bt),
        gt_offsets[..., 0].astype(f32).reshape(bt),
        gt_offsets[..., 1].astype(f32).reshape(bt),
        gt_segments[..., 0].reshape(_B, _NG, 1).astype(f32),
        gt_segments[..., 1].reshape(_B, _NG, 1).astype(f32),
        segments_label.reshape(_B, _NG, 1).astype(jnp.int32),
        segments_mask.reshape(_B, _NG, 1).astype(f32),
        out_rois[..., 1].reshape(_B, 1, _NP).astype(f32),
        out_rois[..., 2].reshape(_B, 1, _NP).astype(f32),
        out_scores.reshape(_B, 1, _NP).astype(f32),
        jnp.transpose(cls_log.astype(f32), (0, 2, 1)),
        jnp.transpose(com_log.astype(f32), (0, 2, 1)),
    )
